# baseline (speedup 1.0000x reference)
import functools

import jax
import jax.numpy as jnp
from jax import lax
from jax.experimental import pallas as pl
from jax.experimental.pallas import tpu as pltpu
from jax.experimental.pallas import tpu_sc as plsc

_F = 8
_D = 32
_B = 16384
_C = 64
_V = 100000
_PACK = 4
_V4 = _V // _PACK
_VT = _V // 128
_TAIL = _V - _VT * 128


@functools.cache
def _build():
  info = plsc.get_sparse_core_info()
  nc, ns = info.num_cores, info.num_subcores
  fpc = _F // nc
  rows_w = _B // ns
  nch = rows_w // _C
  mesh = plsc.VectorSubcoreMesh(core_axis_name="c", subcore_axis_name="s")

  @functools.partial(
      pl.kernel,
      mesh=mesh,
      compiler_params=pltpu.CompilerParams(needs_layout_passes=False),
      out_type=jax.ShapeDtypeStruct((_B, _F * _D), jnp.float32),
      scratch_types=[
          pltpu.HBM((_V4, 128), jnp.float32),
          pltpu.HBM((_V4, 128), jnp.float32),
          pltpu.HBM((_V4, 128), jnp.float32),
          pltpu.HBM((_V4, 128), jnp.float32),
          pltpu.HBM((_V4, 128), jnp.float32),
          pltpu.HBM((_V4, 128), jnp.float32),
          pltpu.HBM((_V4, 128), jnp.float32),
          pltpu.HBM((_V4, 128), jnp.float32),
          pltpu.VMEM((fpc, nch, _C), jnp.int32),
          pltpu.VMEM((fpc, nch, _C), jnp.int32),
          pltpu.VMEM((2, _D, 128), jnp.float32),
          pltpu.VMEM((2, _D, 128), jnp.float32),
          pltpu.VMEM((_PACK, _C, 128), jnp.float32),
          pltpu.VMEM((_C, fpc * _D), jnp.float32),
          pltpu.SemaphoreType.DMA,
          pltpu.SemaphoreType.DMA,
          pltpu.SemaphoreType.DMA,
          pltpu.SemaphoreType.DMA,
          pltpu.SemaphoreType.DMA,
          pltpu.SemaphoreType.DMA,
          pltpu.SemaphoreType.DMA,
          pltpu.SemaphoreType.DMA,
      ],
  )
  def k(j_hbm, rem_hbm, tails, t0, t1, t2, t3, t4, t5, t6, t7, out_hbm,
        p0, p1, p2, p3, p4, p5, p6, p7,
        j_v, rem_v, buf_v, pout_v, rows_v, big_v,
        sr0, sr1, sw0, sw1, sg0, sg1, sg2, sg3):
    tables = (t0, t1, t2, t3, t4, t5, t6, t7)
    packed = (p0, p1, p2, p3, p4, p5, p6, p7)
    srs = (sr0, sr1)
    sws = (sw0, sw1)
    sgs = (sg0, sg1, sg2, sg3)
    c = lax.axis_index("c")
    s = lax.axis_index("s")
    iota16 = lax.iota(jnp.int32, 16)

    def pack_table(table, pk):
      nblk = lax.select(s < _VT % ns, (_VT // ns) + 1, _VT // ns)
      npair = (_VT // ns) // 2
      assert (_VT // ns) % 2 == 0

      def rd(i, pp):
        cc = s + i * ns
        return pltpu.make_async_copy(
            table.at[:, pl.ds(cc * 128, 128)], buf_v.at[pp], srs[pp])

      def wr(i, pp):
        cc = s + i * ns
        return pltpu.make_async_copy(
            pout_v.at[pp], pk.at[pl.ds(cc * _D, _D), :], sws[pp])

      def shuffle(pp):
        def body(jj, _):
          for h in range(8):
            rowv = iota16 + (16 * (h % 2))
            colv = jnp.full((16,), 0, jnp.int32) + (_PACK * jj + h // 2)
            pout_v[pp, jj, pl.ds(h * 16, 16)] = plsc.load_gather(
                buf_v.at[pp], [rowv, colv])
          return 0

        lax.fori_loop(0, _D, body, 0)

      rd(0, 0).start()
      rd(1, 1).start()

      def pair(p, _):
        i0 = 2 * p
        i1 = i0 + 1

        @pl.when(p >= 1)
        def _():
          wr(i0 - 2, 0).wait()

        rd(i0, 0).wait()
        shuffle(0)

        @pl.when(i0 + 2 < nblk)
        def _():
          rd(i0 + 2, 0).start()

        wr(i0, 0).start()

        @pl.when(p >= 1)
        def _():
          wr(i1 - 2, 1).wait()

        rd(i1, 1).wait()
        shuffle(1)

        @pl.when(i1 + 2 < nblk)
        def _():
          rd(i1 + 2, 1).start()

        wr(i1, 1).start()
        return 0

      lax.fori_loop(0, npair, pair, 0)
      last = 2 * npair
      wr(last - 2, 0).wait()
      wr(last - 1, 1).wait()

      @pl.when(last < nblk)
      def _():
        rd(last, 0).wait()
        shuffle(0)
        wr(last, 0).start()
        wr(last, 0).wait()

    def gather_phase(fs, col_off):
      for tt in range(len(fs)):
        pltpu.sync_copy(j_hbm.at[fs[tt], s], j_v.at[tt])
        pltpu.sync_copy(rem_hbm.at[fs[tt], s], rem_v.at[tt])

      def g(q, tt):
        return pltpu.make_async_copy(
            packed[fs[tt]].at[j_v.at[tt, q]], rows_v.at[tt], sgs[tt])

      for tt in range(len(fs)):
        g(0, tt).start()

      def chunk(q, _):
        for tt in range(len(fs)):
          g(q, tt).wait()

          def body(r16, _, tt=tt):
            rbase = r16 * 16
            offs = rem_v[tt, q, pl.ds(rbase, 16)] * _D
            for kk in range(16):
              r = rbase + kk
              off = offs[kk]
              big_v[r, pl.ds(tt * _D, 16)] = rows_v[tt, r, pl.ds(off, 16)]
              big_v[r, pl.ds(tt * _D + 16, 16)] = (
                  rows_v[tt, r, pl.ds(off + 16, 16)])
            return 0

          lax.fori_loop(0, _C // 16, body, 0)

          @pl.when(q + 1 < nch)
          def _(tt=tt):
            g(q + 1, tt).start()
        pltpu.sync_copy(
            big_v,
            out_hbm.at[pl.ds(s * rows_w + q * _C, _C),
                       pl.ds(col_off, len(fs) * _D)])
        return 0

      lax.fori_loop(0, nch, chunk, 0)

    for cbr in range(nc):
      @pl.when(c == cbr)
      def _(cbr=cbr):
        fs = tuple(range(cbr * fpc, (cbr + 1) * fpc))
        for tt in range(fpc):
          pack_table(tables[fs[tt]], packed[fs[tt]])

        @pl.when(s == 0)
        def _(fs=fs):
          for tt in range(fpc):
            pltpu.sync_copy(tails.at[fs[tt]],
                            buf_v.at[0, pl.ds(0, _TAIL // _PACK)])
            pltpu.sync_copy(buf_v.at[0, pl.ds(0, _TAIL // _PACK)],
                            packed[fs[tt]].at[pl.ds(_V4 - _TAIL // _PACK,
                                                    _TAIL // _PACK), :])
        plsc.subcore_barrier()
        gather_phase(fs, cbr * fpc * _D)

  return k, ns, nch


def kernel(f0, f1, f2, f3, f4, f5, f6, f7,
           W_f0, W_f1, W_f2, W_f3, W_f4, W_f5, W_f6, W_f7):
  k, ns, nch = _build()
  ws = (W_f0, W_f1, W_f2, W_f3, W_f4, W_f5, W_f6, W_f7)
  idx = jnp.stack([f0, f1, f2, f3, f4, f5, f6, f7]).astype(jnp.int32)
  j = (idx >> 2).reshape(_F, ns, nch, _C)
  rem = (idx & 3).reshape(_F, ns, nch, _C)
  tails = jnp.stack(
      [w[_VT * 128:].reshape(_TAIL // _PACK, _PACK * _D) for w in ws])
  return k(j, rem, tails, *[w.T for w in ws])


# speedup vs baseline: 1.7574x; 1.7574x over previous
import functools

import jax
import jax.numpy as jnp
from jax import lax
from jax.experimental import pallas as pl
from jax.experimental.pallas import tpu as pltpu
from jax.experimental.pallas import tpu_sc as plsc

_G = 4
_D = 32
_B = 16384
_C = 64
_K = 4
_PACK = 4
_V = 100000


@functools.cache
def _build():
  info = plsc.get_sparse_core_info()
  nc, ns = info.num_cores, info.num_subcores
  nw = nc * ns
  n = _B // nw
  nq = n // _C
  mesh = plsc.VectorSubcoreMesh(core_axis_name="c", subcore_axis_name="s")

  @functools.partial(
      pl.kernel,
      mesh=mesh,
      out_type=jax.ShapeDtypeStruct((_B, _G * _D), jnp.float32),
      scratch_types=[
          pltpu.VMEM((_G, nq, _C), jnp.int32),
          pltpu.VMEM((_G, nq, _C), jnp.int32),
          pltpu.VMEM((_K, _C, 128), jnp.float32),
          pltpu.VMEM((_C, _G * _D), jnp.float32),
          pltpu.SemaphoreType.DMA,
          pltpu.SemaphoreType.DMA,
          pltpu.SemaphoreType.DMA,
          pltpu.SemaphoreType.DMA,
      ],
  )
  def k(j_hbm, rem_hbm, t0, t1, t2, t3, out_hbm,
        j_v, rem_v, rows_v, big_v, sem0, sem1, sem2, sem3):
    tables = (t0, t1, t2, t3)
    sems = (sem0, sem1, sem2, sem3)
    wid = lax.axis_index("c") * ns + lax.axis_index("s")
    base = wid * n

    for f in range(_G):
      pltpu.sync_copy(j_hbm.at[f, wid], j_v.at[f])
      pltpu.sync_copy(rem_hbm.at[f, wid], rem_v.at[f])

    def g(q, f):
      return pltpu.make_async_copy(
          tables[f].at[j_v.at[f, q]], rows_v.at[f % _K], sems[f % _K])

    for f in range(_G):
      g(0, f).start()

    def chunk(q, _):
      for f in range(_G):
        g(q, f).wait()

        def body(r16, _, f=f):
          rbase = r16 * 16
          offs = rem_v[f, q, pl.ds(rbase, 16)] * _D
          for kk in range(16):
            r = rbase + kk
            off = offs[kk]
            big_v[r, pl.ds(f * _D, 16)] = rows_v[f, r, pl.ds(off, 16)]
            big_v[r, pl.ds(f * _D + 16, 16)] = (
                rows_v[f, r, pl.ds(off + 16, 16)])
          return 0

        lax.fori_loop(0, _C // 16, body, 0)

        @pl.when(q + 1 < nq)
        def _(f=f):
          g(q + 1, f).start()
      pltpu.sync_copy(big_v, out_hbm.at[pl.ds(base + q * _C, _C), :])
      return 0

    lax.fori_loop(0, nq, chunk, 0)

  return k, nw, nq


def kernel(f0, f1, f2, f3, f4, f5, f6, f7,
           W_f0, W_f1, W_f2, W_f3, W_f4, W_f5, W_f6, W_f7):
  k, nw, nq = _build()
  ws = (W_f0, W_f1, W_f2, W_f3, W_f4, W_f5, W_f6, W_f7)
  idx = jnp.stack([f0, f1, f2, f3, f4, f5, f6, f7]).astype(jnp.int32)
  j = (idx >> 2).reshape(8, nw, nq, _C)
  rem = (idx & 3).reshape(8, nw, nq, _C)
  halves = []
  for grp in range(2):
    lo = grp * _G
    packed = [ws[lo + t].reshape(_V // _PACK, _D * _PACK) for t in range(_G)]
    halves.append(k(j[lo:lo + _G], rem[lo:lo + _G], *packed))
  return jnp.concatenate(halves, axis=-1)
